# baseline (speedup 1.0000x reference)
import functools

import jax
import jax.numpy as jnp
from jax import lax
from jax.experimental import pallas as pl
from jax.experimental.pallas import tpu as pltpu
from jax.experimental.pallas import tpu_sc as plsc

VOCAB = 50304
D = 768
B, T = 4, 2048
BT = B * T

NC, NS, L = 2, 16, 16
NW = NC * NS
T_PER_W = T // NW
CHUNK = 32
SUB = T_PER_W // CHUNK
NCHUNK = B * SUB
NBUF = 2


@functools.cache
def _make_kernel():
  mesh = plsc.VectorSubcoreMesh(core_axis_name="c", subcore_axis_name="s")

  @functools.partial(
      pl.kernel,
      mesh=mesh,
      out_type=jax.ShapeDtypeStruct((BT, D), jnp.float32),
      scratch_types=[
          pltpu.VMEM((NCHUNK, CHUNK), jnp.int32),
          pltpu.VMEM((T_PER_W, D), jnp.float32),
      ] + [pltpu.VMEM((CHUNK, D), jnp.float32) for _ in range(NBUF)]
        + [pltpu.SemaphoreType.DMA for _ in range(2 * NBUF)],
  )
  def emb_kernel(idx_hbm, wte_hbm, wpe_hbm, out_hbm, idx_v, wpe_v,
                 r0, r1, g0, g1, s0, s1):
    rows = (r0, r1)
    gsem = (g0, g1)
    ssem = (s0, s1)
    wid = lax.axis_index("s") * NC + lax.axis_index("c")
    t0 = wid * T_PER_W

    for b in range(B):
      pltpu.sync_copy(idx_hbm.at[pl.ds(b * (T // CHUNK) + wid * SUB, SUB)],
                      idx_v.at[pl.ds(b * SUB, SUB)])
    pltpu.sync_copy(wpe_hbm.at[pl.ds(t0, T_PER_W)], wpe_v)

    def start_gather(k, p):
      return pltpu.async_copy(wte_hbm.at[idx_v.at[k]], rows[p], gsem[p])

    for p in range(NBUF):
      start_gather(p, p)

    def batch_body(g, _):
      for p in range(NBUF):
        pltpu.make_async_copy(wte_hbm.at[idx_v.at[p]], rows[p],
                              gsem[p]).wait()

        def add_row(i, _, p=p):
          G = 8
          for q in range(D // L // G):
            sls = [pl.ds((q * G + u) * L, L) for u in range(G)]
            vals = [wpe_v[p * CHUNK + i, sl] for sl in sls]
            for sl, v in zip(sls, vals):
              plsc.addupdate(rows[p].at[i, sl], v)
          return 0

        lax.fori_loop(0, CHUNK, add_row, 0)
        row0 = g * T + t0 + p * CHUNK
        pltpu.async_copy(rows[p], out_hbm.at[pl.ds(row0, CHUNK)], ssem[p])
      for p in range(NBUF):
        @pl.when(g < B - 1)
        def _(g=g, p=p):
          pltpu.make_async_copy(rows[p], out_hbm.at[pl.ds(t0, CHUNK)],
                                ssem[p]).wait()
          start_gather(SUB * (g + 1) + p, p)
      return 0

    lax.fori_loop(0, B, batch_body, 0)
    for p in range(NBUF):
      pltpu.make_async_copy(rows[p], out_hbm.at[pl.ds(t0, CHUNK)],
                            ssem[p]).wait()

  return emb_kernel


def kernel(idx, targets, wte, wpe):
  idx2 = idx.astype(jnp.int32).reshape(BT // CHUNK, CHUNK)
  x = _make_kernel()(idx2, wte, wpe)
  return x.reshape(B, T, D), targets
